# baseline (speedup 1.0000x reference)
import functools

import jax
import jax.numpy as jnp
from jax import lax
from jax.experimental import pallas as pl
from jax.experimental.pallas import tpu as pltpu
from jax.experimental.pallas import tpu_sc as plsc

N = 10000
E = 320000
D = 128
NC = 2
NS = 16
NW = NC * NS
EC = E // NW
G = 80
NB = EC // G
RPT = 624
TAIL = N - NS * RPT
INV_SQRT_D = 1.0 / float(D) ** 0.5


def _qkv_tc(x, Wq, Wk, Wv, g1, b1):
    R = 1000

    def body(x_ref, wq_ref, wk_ref, wv_ref, g_ref, b_ref, q_ref, k_ref, vl_ref, vr_ref):
        xb = x_ref[...]
        mu = jnp.mean(xb, axis=-1, keepdims=True)
        var = jnp.mean((xb - mu) ** 2, axis=-1, keepdims=True)
        h = (xb - mu) / jnp.sqrt(var + 1e-5) * g_ref[...] + b_ref[...]
        q_ref[...] = jnp.dot(h, wq_ref[...], preferred_element_type=jnp.float32)
        k_ref[...] = jnp.dot(h, wk_ref[...], preferred_element_type=jnp.float32)
        vfull = jnp.dot(h, wv_ref[...], preferred_element_type=jnp.float32)
        vl_ref[...] = vfull[:, :DH]
        vr_ref[...] = vfull[:, DH:]

    out = pl.pallas_call(
        body,
        grid=(N // R,),
        in_specs=[
            pl.BlockSpec((R, D), lambda i: (i, 0)),
            pl.BlockSpec((D, D), lambda i: (0, 0)),
            pl.BlockSpec((D, D), lambda i: (0, 0)),
            pl.BlockSpec((D, D), lambda i: (0, 0)),
            pl.BlockSpec((1, D), lambda i: (0, 0)),
            pl.BlockSpec((1, D), lambda i: (0, 0)),
        ],
        out_specs=[
            pl.BlockSpec((R, D), lambda i: (i, 0)),
            pl.BlockSpec((R, D), lambda i: (i, 0)),
            pl.BlockSpec((R, DH), lambda i: (i, 0)),
            pl.BlockSpec((R, DH), lambda i: (i, 0)),
        ],
        out_shape=[jax.ShapeDtypeStruct((N, D), jnp.float32)] * 2
        + [jax.ShapeDtypeStruct((N, DH), jnp.float32)] * 2,
    )(x, Wq, Wk, Wv, g1.reshape(1, D), b1.reshape(1, D))
    return out


def _edge_logits_sc(q, k, src, dst):
    mesh = plsc.VectorSubcoreMesh(
        core_axis_name="c", subcore_axis_name="s", num_cores=NC, num_subcores=NS
    )

    @functools.partial(
        pl.kernel,
        out_type=[
            jax.ShapeDtypeStruct((NW, NB, G), jnp.float32),
            jax.ShapeDtypeStruct((NW, N), jnp.float32),
        ],
        mesh=mesh,
        compiler_params=pltpu.CompilerParams(needs_layout_passes=False),
        scratch_types=[
            pltpu.VMEM((NB, G), jnp.int32),
            pltpu.VMEM((NB, G), jnp.int32),
            pltpu.VMEM((NB, G), jnp.float32),
            pltpu.VMEM((N,), jnp.float32),
            pltpu.VMEM((3, G, D), jnp.float32),
            pltpu.VMEM((3, G, D), jnp.float32),
            pltpu.SemaphoreType.DMA,
            pltpu.SemaphoreType.DMA,
        ],
    )
    def kern(q_hbm, k_hbm, src_hbm, dst_hbm, e_hbm, mpart_hbm,
             dst_v, src_v, e_v, m_v, qb, kb, semq, semk):
        cid = lax.axis_index("c")
        sid = lax.axis_index("s")
        wid = cid * NS + sid

        pltpu.sync_copy(dst_hbm.at[wid], dst_v)
        pltpu.sync_copy(src_hbm.at[wid], src_v)

        neg = jnp.full((16,), -1e30, dtype=jnp.float32)

        def init_m(g, carry):
            m_v[pl.ds(g * 16, 16)] = neg
            return carry

        lax.fori_loop(0, N // 16, init_m, 0)

        iota16 = lax.iota(jnp.int32, 16)

        def issue(b, pb):
            pltpu.async_copy(q_hbm.at[dst_v.at[b]], qb.at[pb], semq)
            pltpu.async_copy(k_hbm.at[src_v.at[b]], kb.at[pb], semk)

        for pre in range(2):
            issue(pre, pre)

        def block_body(b, carry):
            pb = lax.rem(b, 3)
            pltpu.make_async_copy(q_hbm.at[dst_v.at[b]], qb.at[pb], semq).wait()
            pltpu.make_async_copy(k_hbm.at[src_v.at[b]], kb.at[pb], semk).wait()

            @pl.when(b + 2 < NB)
            def _():
                issue(b + 2, lax.rem(b + 2, 3))

            for g in range(G // 16):
                rows = g * 16 + iota16

                def c_body(c, acc):
                    colv = jnp.zeros((16,), jnp.int32) + c
                    qv = plsc.load_gather(qb.at[pb], [rows, colv])
                    kv = plsc.load_gather(kb.at[pb], [rows, colv])
                    return acc + qv * kv

                acc = lax.fori_loop(0, D, c_body,
                                    jnp.zeros((16,), jnp.float32), unroll=8)
                e16 = acc * INV_SQRT_D
                e_v[b, pl.ds(g * 16, 16)] = e16
                dv = dst_v[b, pl.ds(g * 16, 16)]
                cnt, _ = plsc.scan_count(dv)
                maxc = jnp.max(cnt)

                def m_pass(pp, inner):
                    mg = plsc.load_gather(m_v, [dv])
                    nm = jnp.maximum(mg, e16)
                    plsc.store_scatter(m_v, [dv], nm, mask=cnt == pp)
                    return inner

                lax.fori_loop(0, maxc + 1, m_pass, 0)
            return carry

        lax.fori_loop(0, NB, block_body, 0)

        pltpu.sync_copy(e_v, e_hbm.at[wid])
        pltpu.sync_copy(m_v, mpart_hbm.at[wid])

    return kern(q, k, src, dst)


def _merge_max_tc(mpart):
    def body(mp_ref, m_ref):
        m_ref[...] = jnp.max(mp_ref[...], axis=0, keepdims=True)

    return pl.pallas_call(
        body,
        out_shape=jax.ShapeDtypeStruct((1, N), jnp.float32),
    )(mpart)


DH = 64


def _scatter_sc(ew, m, vh, src, dst, first):
    mesh = plsc.VectorSubcoreMesh(
        core_axis_name="c", subcore_axis_name="s", num_cores=NC, num_subcores=NS
    )

    out_type = [jax.ShapeDtypeStruct((NC, N, DH), jnp.float32)]
    if first:
        out_type += [
            jax.ShapeDtypeStruct((NW, NB, G), jnp.float32),
            jax.ShapeDtypeStruct((NW, N), jnp.float32),
        ]

    @functools.partial(
        pl.kernel,
        out_type=out_type,
        mesh=mesh,
        compiler_params=pltpu.CompilerParams(
            needs_layout_passes=False, use_tc_tiling_on_sc=False),
        scratch_types=[
            pltpu.VMEM((NB, G), jnp.int32),
            pltpu.VMEM((NB, G), jnp.int32),
            pltpu.VMEM((NB, G), jnp.float32),
            pltpu.VMEM((N,), jnp.float32),
            pltpu.VMEM((N,), jnp.float32),
            pltpu.VMEM((2, G, DH), jnp.float32),
            pltpu.VMEM((G, DH), jnp.float32),
            pltpu.VMEM((208, DH), jnp.float32),
            pltpu.VMEM_SHARED((N, DH), jnp.float32),
            pltpu.SemaphoreType.DMA,
        ],
    )
    def kern(*args):
        if first:
            (ew_hbm, m_hbm, v_hbm, src_hbm, dst_hbm, s_hbm, w_hbm, dpart_hbm,
             dst_v, src_v, w_v, m_v, d_v, vb, vx, zb, s_sh, sem) = args
        else:
            (ew_hbm, v_hbm, src_hbm, dst_hbm, s_hbm,
             dst_v, src_v, w_v, m_v, d_v, vb, vx, zb, s_sh, sem) = args
        cid = lax.axis_index("c")
        sid = lax.axis_index("s")
        wid = cid * NS + sid

        pltpu.sync_copy(dst_hbm.at[wid], dst_v)
        pltpu.sync_copy(src_hbm.at[wid], src_v)
        pltpu.sync_copy(ew_hbm.at[wid], w_v)

        iota16 = lax.iota(jnp.int32, 16)
        zeros16 = jnp.zeros((16,), jnp.float32)

        if first:
            pltpu.sync_copy(m_hbm.at[0], m_v)

            def init_d(g, carry):
                d_v[pl.ds(g * 16, 16)] = zeros16
                return carry

            lax.fori_loop(0, N // 16, init_d, 0)

            def w_body(t, carry):
                b = t // (G // 16)
                g = t % (G // 16)
                dv = dst_v[b, pl.ds(g * 16, 16)]
                mg = plsc.load_gather(m_v, [dv])
                w16 = jnp.exp(w_v[b, pl.ds(g * 16, 16)] - mg)
                w_v[b, pl.ds(g * 16, 16)] = w16
                cnt, _ = plsc.scan_count(dv)
                maxc = jnp.max(cnt)

                def d_pass(pp, inner):
                    dg = plsc.load_gather(d_v, [dv])
                    plsc.store_scatter(d_v, [dv], dg + w16, mask=cnt == pp)
                    return inner

                lax.fori_loop(0, maxc + 1, d_pass, 0)
                return carry

            lax.fori_loop(0, NB * (G // 16), w_body, 0)

        def z_body(t, carry):
            zb[t // (DH // 16), pl.ds((t % (DH // 16)) * 16, 16)] = zeros16
            return carry

        lax.fori_loop(0, 208 * (DH // 16), z_body, 0)
        for r in range(RPT // 208):
            pltpu.sync_copy(zb, s_sh.at[pl.ds(sid * RPT + r * 208, 208)])

        @pl.when(sid == 0)
        def _():
            pltpu.sync_copy(zb.at[pl.ds(0, TAIL)],
                            s_sh.at[pl.ds(NS * RPT, TAIL)])

        plsc.subcore_barrier()

        pltpu.async_copy(v_hbm.at[src_v.at[0]], vb.at[0], sem)

        def do_block(b, pb):
            pltpu.make_async_copy(v_hbm.at[src_v.at[b]], vb.at[pb], sem).wait()

            @pl.when(b + 1 < NB)
            def _():
                pltpu.async_copy(v_hbm.at[src_v.at[b + 1]], vb.at[1 - pb], sem)

            def group_body(g, inner):
                wg = w_v[b, pl.ds(g * 16, 16)]
                for j16 in range(16):
                    j = g * 16 + j16
                    wj = wg[j16]
                    for c in range(DH // 16):
                        vx[j, pl.ds(c * 16, 16)] = (
                            vb[pb, j, pl.ds(c * 16, 16)] * wj)
                return inner

            lax.fori_loop(0, G // 16, group_body, 0)
            pltpu.sync_copy(vx, s_sh.at[dst_v.at[b]], add=True)

        def pair_body(t, carry):
            do_block(2 * t, 0)
            do_block(2 * t + 1, 1)
            return carry

        lax.fori_loop(0, NB // 2, pair_body, 0)
        do_block(NB - 1, 0)

        plsc.subcore_barrier()

        pltpu.sync_copy(
            s_sh.at[pl.ds(sid * RPT, RPT)],
            s_hbm.at[cid].at[pl.ds(sid * RPT, RPT)],
        )

        @pl.when(sid == 0)
        def _():
            pltpu.sync_copy(s_sh.at[pl.ds(NS * RPT, TAIL)],
                            s_hbm.at[cid].at[pl.ds(NS * RPT, TAIL)])

        if first:
            pltpu.sync_copy(w_v, w_hbm.at[wid])
            pltpu.sync_copy(d_v, dpart_hbm.at[wid])

    if first:
        return kern(ew, m, vh, src, dst)
    return kern(ew, vh, src, dst)


def _merge_den_tc(dpart):
    def body(dp_ref, d_ref):
        d_ref[...] = jnp.sum(dp_ref[...], axis=0, keepdims=True)

    return pl.pallas_call(
        body,
        out_shape=jax.ShapeDtypeStruct((1, N), jnp.float32),
    )(dpart)


def _finish_tc(x, S0, S1, den, Wo, g2, b2):
    R = 1000

    def body(x_ref, s0_ref, s1_ref, den_ref, wo_ref, g_ref, b_ref, y_ref):
        rden = 1.0 / (den_ref[...] + 1e-16)
        aggl = (s0_ref[0] + s0_ref[1]) * rden
        aggr = (s1_ref[0] + s1_ref[1]) * rden
        out = (
            jnp.dot(aggl, wo_ref[pl.ds(0, DH), :],
                    preferred_element_type=jnp.float32)
            + jnp.dot(aggr, wo_ref[pl.ds(DH, DH), :],
                      preferred_element_type=jnp.float32)
        )
        mu = jnp.mean(out, axis=-1, keepdims=True)
        var = jnp.mean((out - mu) ** 2, axis=-1, keepdims=True)
        ln = (out - mu) / jnp.sqrt(var + 1e-5) * g_ref[...] + b_ref[...]
        y_ref[...] = x_ref[...] + jnp.maximum(ln, 0.0)

    return pl.pallas_call(
        body,
        grid=(N // R,),
        in_specs=[
            pl.BlockSpec((R, D), lambda i: (i, 0)),
            pl.BlockSpec((NC, R, DH), lambda i: (0, i, 0)),
            pl.BlockSpec((NC, R, DH), lambda i: (0, i, 0)),
            pl.BlockSpec((R, 1), lambda i: (i, 0)),
            pl.BlockSpec((D, D), lambda i: (0, 0)),
            pl.BlockSpec((1, D), lambda i: (0, 0)),
            pl.BlockSpec((1, D), lambda i: (0, 0)),
        ],
        out_specs=pl.BlockSpec((R, D), lambda i: (i, 0)),
        out_shape=jax.ShapeDtypeStruct((N, D), jnp.float32),
    )(x, S0, S1, den, Wo, g2.reshape(1, D), b2.reshape(1, D))


def kernel(x, edge_index, Wq, Wk, Wv, Wo, g1, b1, g2, b2):
    src = edge_index[0].reshape(NW, NB, G)
    dst = edge_index[1].reshape(NW, NB, G)
    q, k, vl, vr = _qkv_tc(x, Wq, Wk, Wv, g1, b1)
    e, mpart = _edge_logits_sc(q, k, src, dst)
    m = _merge_max_tc(mpart)
    S0, w, dpart = _scatter_sc(e, m, vl, src, dst, first=True)
    (S1,) = _scatter_sc(w, None, vr, src, dst, first=False)
    den = _merge_den_tc(dpart).reshape(N, 1)
    return _finish_tc(x, S0, S1, den, Wo, g2, b2)


# speedup vs baseline: 1.9590x; 1.9590x over previous
import functools

import jax
import jax.numpy as jnp
from jax import lax
from jax.experimental import pallas as pl
from jax.experimental.pallas import tpu as pltpu
from jax.experimental.pallas import tpu_sc as plsc

N = 10000
E = 320000
D = 128
NC = 2
NS = 16
NW = NC * NS
EC = E // NW
G = 80
NB = EC // G
RPT = 624
TAIL = N - NS * RPT
INV_SQRT_D = 1.0 / float(D) ** 0.5


def _qkv_tc(x, Wq, Wk, Wv, g1, b1):
    R = 1000

    def body(x_ref, wq_ref, wk_ref, wv_ref, g_ref, b_ref, q_ref, k_ref, vl_ref, vr_ref):
        xb = x_ref[...]
        mu = jnp.mean(xb, axis=-1, keepdims=True)
        var = jnp.mean((xb - mu) ** 2, axis=-1, keepdims=True)
        h = (xb - mu) / jnp.sqrt(var + 1e-5) * g_ref[...] + b_ref[...]
        q_ref[...] = jnp.dot(h, wq_ref[...], preferred_element_type=jnp.float32)
        k_ref[...] = jnp.dot(h, wk_ref[...], preferred_element_type=jnp.float32)
        vfull = jnp.dot(h, wv_ref[...], preferred_element_type=jnp.float32)
        vl_ref[...] = vfull[:, :DH]
        vr_ref[...] = vfull[:, DH:]

    out = pl.pallas_call(
        body,
        grid=(N // R,),
        in_specs=[
            pl.BlockSpec((R, D), lambda i: (i, 0)),
            pl.BlockSpec((D, D), lambda i: (0, 0)),
            pl.BlockSpec((D, D), lambda i: (0, 0)),
            pl.BlockSpec((D, D), lambda i: (0, 0)),
            pl.BlockSpec((1, D), lambda i: (0, 0)),
            pl.BlockSpec((1, D), lambda i: (0, 0)),
        ],
        out_specs=[
            pl.BlockSpec((R, D), lambda i: (i, 0)),
            pl.BlockSpec((R, D), lambda i: (i, 0)),
            pl.BlockSpec((R, DH), lambda i: (i, 0)),
            pl.BlockSpec((R, DH), lambda i: (i, 0)),
        ],
        out_shape=[jax.ShapeDtypeStruct((N, D), jnp.float32)] * 2
        + [jax.ShapeDtypeStruct((N, DH), jnp.float32)] * 2,
    )(x, Wq, Wk, Wv, g1.reshape(1, D), b1.reshape(1, D))
    return out


def _edge_logits_sc(q, k, src, dst):
    mesh = plsc.VectorSubcoreMesh(
        core_axis_name="c", subcore_axis_name="s", num_cores=NC, num_subcores=NS
    )

    @functools.partial(
        pl.kernel,
        out_type=[
            jax.ShapeDtypeStruct((NW, NB, G), jnp.float32),
            jax.ShapeDtypeStruct((NW, N), jnp.float32),
        ],
        mesh=mesh,
        compiler_params=pltpu.CompilerParams(needs_layout_passes=False),
        scratch_types=[
            pltpu.VMEM((NB, G), jnp.int32),
            pltpu.VMEM((NB, G), jnp.int32),
            pltpu.VMEM((NB, G), jnp.float32),
            pltpu.VMEM((N,), jnp.float32),
            pltpu.VMEM((3, G, D), jnp.float32),
            pltpu.VMEM((3, G, D), jnp.float32),
            pltpu.SemaphoreType.DMA,
            pltpu.SemaphoreType.DMA,
        ],
    )
    def kern(q_hbm, k_hbm, src_hbm, dst_hbm, e_hbm, mpart_hbm,
             dst_v, src_v, e_v, m_v, qb, kb, semq, semk):
        cid = lax.axis_index("c")
        sid = lax.axis_index("s")
        wid = cid * NS + sid

        pltpu.sync_copy(dst_hbm.at[wid], dst_v)
        pltpu.sync_copy(src_hbm.at[wid], src_v)

        neg = jnp.full((16,), -1e30, dtype=jnp.float32)

        def init_m(g, carry):
            m_v[pl.ds(g * 16, 16)] = neg
            return carry

        lax.fori_loop(0, N // 16, init_m, 0)

        iota16 = lax.iota(jnp.int32, 16)

        def issue(b, pb):
            pltpu.async_copy(q_hbm.at[dst_v.at[b]], qb.at[pb], semq)
            pltpu.async_copy(k_hbm.at[src_v.at[b]], kb.at[pb], semk)

        for pre in range(2):
            issue(pre, pre)

        def block_body(b, carry):
            pb = lax.rem(b, 3)
            e_row = e_v.at[b]
            pltpu.make_async_copy(q_hbm.at[dst_v.at[b]], qb.at[pb], semq).wait()
            pltpu.make_async_copy(k_hbm.at[src_v.at[b]], kb.at[pb], semk).wait()

            @pl.when(b + 2 < NB)
            def _():
                issue(b + 2, lax.rem(b + 2, 3))

            for g in range(G // 16):
                eidx = g * 16 + iota16
                for j16 in range(16):
                    j = g * 16 + j16
                    acc = qb[pb, j, pl.ds(0, 16)] * kb[pb, j, pl.ds(0, 16)]
                    for c in range(1, D // 16):
                        acc += (qb[pb, j, pl.ds(c * 16, 16)]
                                * kb[pb, j, pl.ds(c * 16, 16)])
                    ej = jnp.sum(acc) * INV_SQRT_D
                    plsc.store_scatter(e_row, [eidx],
                                       jnp.zeros((16,), jnp.float32) + ej,
                                       mask=iota16 == j16)
                e16 = e_row[pl.ds(g * 16, 16)]
                dv = dst_v[b, pl.ds(g * 16, 16)]
                cnt, _ = plsc.scan_count(dv)
                maxc = jnp.max(cnt)

                def m_pass(pp, inner):
                    mg = plsc.load_gather(m_v, [dv])
                    nm = jnp.maximum(mg, e16)
                    plsc.store_scatter(m_v, [dv], nm, mask=cnt == pp)
                    return inner

                lax.fori_loop(0, maxc + 1, m_pass, 0)
            return carry

        lax.fori_loop(0, NB, block_body, 0)

        pltpu.sync_copy(e_v, e_hbm.at[wid])
        pltpu.sync_copy(m_v, mpart_hbm.at[wid])

    return kern(q, k, src, dst)


def _merge_max_tc(mpart):
    def body(mp_ref, m_ref):
        m_ref[...] = jnp.max(mp_ref[...], axis=0, keepdims=True)

    return pl.pallas_call(
        body,
        out_shape=jax.ShapeDtypeStruct((1, N), jnp.float32),
    )(mpart)


DH = 64


def _scatter_sc(ew, m, vh, src, dst, first):
    mesh = plsc.VectorSubcoreMesh(
        core_axis_name="c", subcore_axis_name="s", num_cores=NC, num_subcores=NS
    )

    out_type = [jax.ShapeDtypeStruct((NC, N, DH), jnp.float32)]
    if first:
        out_type += [
            jax.ShapeDtypeStruct((NW, NB, G), jnp.float32),
            jax.ShapeDtypeStruct((NW, N), jnp.float32),
        ]

    @functools.partial(
        pl.kernel,
        out_type=out_type,
        mesh=mesh,
        compiler_params=pltpu.CompilerParams(
            needs_layout_passes=False, use_tc_tiling_on_sc=False),
        scratch_types=[
            pltpu.VMEM((NB, G), jnp.int32),
            pltpu.VMEM((NB, G), jnp.int32),
            pltpu.VMEM((NB, G), jnp.float32),
            pltpu.VMEM((N,), jnp.float32),
            pltpu.VMEM((N,), jnp.float32),
            pltpu.VMEM((3, G, DH), jnp.float32),
            pltpu.VMEM((G, DH), jnp.float32),
            pltpu.VMEM((208, DH), jnp.float32),
            pltpu.VMEM_SHARED((N, DH), jnp.float32),
            pltpu.SemaphoreType.DMA,
        ],
    )
    def kern(*args):
        if first:
            (ew_hbm, m_hbm, v_hbm, src_hbm, dst_hbm, s_hbm, w_hbm, dpart_hbm,
             dst_v, src_v, w_v, m_v, d_v, vb, vx, zb, s_sh, sem) = args
        else:
            (ew_hbm, v_hbm, src_hbm, dst_hbm, s_hbm,
             dst_v, src_v, w_v, m_v, d_v, vb, vx, zb, s_sh, sem) = args
        cid = lax.axis_index("c")
        sid = lax.axis_index("s")
        wid = cid * NS + sid

        pltpu.sync_copy(dst_hbm.at[wid], dst_v)
        pltpu.sync_copy(src_hbm.at[wid], src_v)
        pltpu.sync_copy(ew_hbm.at[wid], w_v)

        iota16 = lax.iota(jnp.int32, 16)
        zeros16 = jnp.zeros((16,), jnp.float32)

        if first:
            pltpu.sync_copy(m_hbm.at[0], m_v)

            def init_d(g, carry):
                d_v[pl.ds(g * 16, 16)] = zeros16
                return carry

            lax.fori_loop(0, N // 16, init_d, 0)

            def w_body(t, carry):
                b = t // (G // 16)
                g = t % (G // 16)
                dv = dst_v[b, pl.ds(g * 16, 16)]
                mg = plsc.load_gather(m_v, [dv])
                w16 = jnp.exp(w_v[b, pl.ds(g * 16, 16)] - mg)
                w_v[b, pl.ds(g * 16, 16)] = w16
                cnt, _ = plsc.scan_count(dv)
                maxc = jnp.max(cnt)

                def d_pass(pp, inner):
                    dg = plsc.load_gather(d_v, [dv])
                    plsc.store_scatter(d_v, [dv], dg + w16, mask=cnt == pp)
                    return inner

                lax.fori_loop(0, maxc + 1, d_pass, 0)
                return carry

            lax.fori_loop(0, NB * (G // 16), w_body, 0)

        def z_body(t, carry):
            zb[t // (DH // 16), pl.ds((t % (DH // 16)) * 16, 16)] = zeros16
            return carry

        lax.fori_loop(0, 208 * (DH // 16), z_body, 0)
        for r in range(RPT // 208):
            pltpu.sync_copy(zb, s_sh.at[pl.ds(sid * RPT + r * 208, 208)])

        @pl.when(sid == 0)
        def _():
            pltpu.sync_copy(zb.at[pl.ds(0, TAIL)],
                            s_sh.at[pl.ds(NS * RPT, TAIL)])

        plsc.subcore_barrier()

        pltpu.async_copy(v_hbm.at[src_v.at[0]], vb.at[0], sem)
        pltpu.async_copy(v_hbm.at[src_v.at[1]], vb.at[1], sem)

        def do_block(b, pb):
            pltpu.make_async_copy(v_hbm.at[src_v.at[b]], vb.at[pb], sem).wait()

            @pl.when(b + 2 < NB)
            def _():
                pltpu.async_copy(
                    v_hbm.at[src_v.at[b + 2]], vb.at[lax.rem(b + 2, 3)], sem)

            def group_body(g, inner):
                wg = w_v[b, pl.ds(g * 16, 16)]
                for j16 in range(16):
                    j = g * 16 + j16
                    wj = wg[j16]
                    for c in range(DH // 16):
                        vx[j, pl.ds(c * 16, 16)] = (
                            vb[pb, j, pl.ds(c * 16, 16)] * wj)
                return inner

            lax.fori_loop(0, G // 16, group_body, 0)
            pltpu.sync_copy(vx, s_sh.at[dst_v.at[b]], add=True)

        def tri_body(t, carry):
            do_block(3 * t, 0)
            do_block(3 * t + 1, 1)
            do_block(3 * t + 2, 2)
            return carry

        lax.fori_loop(0, NB // 3, tri_body, 0)
        do_block(NB - 2, 0)
        do_block(NB - 1, 1)

        plsc.subcore_barrier()

        pltpu.sync_copy(
            s_sh.at[pl.ds(sid * RPT, RPT)],
            s_hbm.at[cid].at[pl.ds(sid * RPT, RPT)],
        )

        @pl.when(sid == 0)
        def _():
            pltpu.sync_copy(s_sh.at[pl.ds(NS * RPT, TAIL)],
                            s_hbm.at[cid].at[pl.ds(NS * RPT, TAIL)])

        if first:
            pltpu.sync_copy(w_v, w_hbm.at[wid])
            pltpu.sync_copy(d_v, dpart_hbm.at[wid])

    if first:
        return kern(ew, m, vh, src, dst)
    return kern(ew, vh, src, dst)


def _merge_den_tc(dpart):
    def body(dp_ref, d_ref):
        d_ref[...] = jnp.sum(dp_ref[...], axis=0, keepdims=True)

    return pl.pallas_call(
        body,
        out_shape=jax.ShapeDtypeStruct((1, N), jnp.float32),
    )(dpart)


def _finish_tc(x, S0, S1, den, Wo, g2, b2):
    R = 1000

    def body(x_ref, s0_ref, s1_ref, den_ref, wo_ref, g_ref, b_ref, y_ref):
        rden = 1.0 / (den_ref[...] + 1e-16)
        aggl = (s0_ref[0] + s0_ref[1]) * rden
        aggr = (s1_ref[0] + s1_ref[1]) * rden
        out = (
            jnp.dot(aggl, wo_ref[pl.ds(0, DH), :],
                    preferred_element_type=jnp.float32)
            + jnp.dot(aggr, wo_ref[pl.ds(DH, DH), :],
                      preferred_element_type=jnp.float32)
        )
        mu = jnp.mean(out, axis=-1, keepdims=True)
        var = jnp.mean((out - mu) ** 2, axis=-1, keepdims=True)
        ln = (out - mu) / jnp.sqrt(var + 1e-5) * g_ref[...] + b_ref[...]
        y_ref[...] = x_ref[...] + jnp.maximum(ln, 0.0)

    return pl.pallas_call(
        body,
        grid=(N // R,),
        in_specs=[
            pl.BlockSpec((R, D), lambda i: (i, 0)),
            pl.BlockSpec((NC, R, DH), lambda i: (0, i, 0)),
            pl.BlockSpec((NC, R, DH), lambda i: (0, i, 0)),
            pl.BlockSpec((R, 1), lambda i: (i, 0)),
            pl.BlockSpec((D, D), lambda i: (0, 0)),
            pl.BlockSpec((1, D), lambda i: (0, 0)),
            pl.BlockSpec((1, D), lambda i: (0, 0)),
        ],
        out_specs=pl.BlockSpec((R, D), lambda i: (i, 0)),
        out_shape=jax.ShapeDtypeStruct((N, D), jnp.float32),
    )(x, S0, S1, den, Wo, g2.reshape(1, D), b2.reshape(1, D))


def kernel(x, edge_index, Wq, Wk, Wv, Wo, g1, b1, g2, b2):
    src = edge_index[0].reshape(NW, NB, G)
    dst = edge_index[1].reshape(NW, NB, G)
    q, k, vl, vr = _qkv_tc(x, Wq, Wk, Wv, g1, b1)
    e, mpart = _edge_logits_sc(q, k, src, dst)
    m = _merge_max_tc(mpart)
    S0, w, dpart = _scatter_sc(e, m, vl, src, dst, first=True)
    (S1,) = _scatter_sc(w, None, vr, src, dst, first=False)
    den = _merge_den_tc(dpart).reshape(N, 1)
    return _finish_tc(x, S0, S1, den, Wo, g2, b2)


# speedup vs baseline: 2.4828x; 1.2674x over previous
import functools

import jax
import jax.numpy as jnp
from jax import lax
from jax.experimental import pallas as pl
from jax.experimental.pallas import tpu as pltpu
from jax.experimental.pallas import tpu_sc as plsc

N = 10000
E = 320000
D = 128
NC = 2
NS = 16
NW = NC * NS
EC = E // NW
G = 80
NB = EC // G
RPT = 624
TAIL = N - NS * RPT
INV_SQRT_D = 1.0 / float(D) ** 0.5


def _qkv_tc(x, Wq, Wk, Wv, g1, b1):
    R = 1000

    def body(x_ref, wq_ref, wk_ref, wv_ref, g_ref, b_ref, q_ref, k_ref, vl_ref, vr_ref):
        xb = x_ref[...]
        mu = jnp.mean(xb, axis=-1, keepdims=True)
        var = jnp.mean((xb - mu) ** 2, axis=-1, keepdims=True)
        h = (xb - mu) / jnp.sqrt(var + 1e-5) * g_ref[...] + b_ref[...]
        q_ref[...] = jnp.dot(h, wq_ref[...], preferred_element_type=jnp.float32)
        k_ref[...] = jnp.dot(h, wk_ref[...], preferred_element_type=jnp.float32)
        vfull = jnp.dot(h, wv_ref[...], preferred_element_type=jnp.float32)
        vl_ref[...] = vfull[:, :DH]
        vr_ref[...] = vfull[:, DH:]

    out = pl.pallas_call(
        body,
        grid=(N // R,),
        in_specs=[
            pl.BlockSpec((R, D), lambda i: (i, 0)),
            pl.BlockSpec((D, D), lambda i: (0, 0)),
            pl.BlockSpec((D, D), lambda i: (0, 0)),
            pl.BlockSpec((D, D), lambda i: (0, 0)),
            pl.BlockSpec((1, D), lambda i: (0, 0)),
            pl.BlockSpec((1, D), lambda i: (0, 0)),
        ],
        out_specs=[
            pl.BlockSpec((R, D), lambda i: (i, 0)),
            pl.BlockSpec((R, D), lambda i: (i, 0)),
            pl.BlockSpec((R, DH), lambda i: (i, 0)),
            pl.BlockSpec((R, DH), lambda i: (i, 0)),
        ],
        out_shape=[jax.ShapeDtypeStruct((N, D), jnp.float32)] * 2
        + [jax.ShapeDtypeStruct((N, DH), jnp.float32)] * 2,
    )(x, Wq, Wk, Wv, g1.reshape(1, D), b1.reshape(1, D))
    return out


def _edge_logits_sc(q, k, src, dst):
    mesh = plsc.VectorSubcoreMesh(
        core_axis_name="c", subcore_axis_name="s", num_cores=NC, num_subcores=NS
    )

    @functools.partial(
        pl.kernel,
        out_type=[
            jax.ShapeDtypeStruct((NW, NB, G), jnp.float32),
            jax.ShapeDtypeStruct((NW, N), jnp.float32),
        ],
        mesh=mesh,
        compiler_params=pltpu.CompilerParams(needs_layout_passes=False),
        scratch_types=[
            pltpu.VMEM((NB, G), jnp.int32),
            pltpu.VMEM((NB, G), jnp.int32),
            pltpu.VMEM((NB, G), jnp.float32),
            pltpu.VMEM((N,), jnp.float32),
            pltpu.VMEM((3, G, D), jnp.float32),
            pltpu.VMEM((3, G, D), jnp.float32),
            pltpu.SemaphoreType.DMA,
            pltpu.SemaphoreType.DMA,
        ],
    )
    def kern(q_hbm, k_hbm, src_hbm, dst_hbm, e_hbm, mpart_hbm,
             dst_v, src_v, e_v, m_v, qb, kb, semq, semk):
        cid = lax.axis_index("c")
        sid = lax.axis_index("s")
        wid = cid * NS + sid

        pltpu.sync_copy(dst_hbm.at[wid], dst_v)
        pltpu.sync_copy(src_hbm.at[wid], src_v)

        neg = jnp.full((16,), -1e30, dtype=jnp.float32)

        def init_m(g, carry):
            m_v[pl.ds(g * 16, 16)] = neg
            return carry

        lax.fori_loop(0, N // 16, init_m, 0)

        iota16 = lax.iota(jnp.int32, 16)

        def issue(b, pb):
            pltpu.async_copy(q_hbm.at[dst_v.at[b]], qb.at[pb], semq)
            pltpu.async_copy(k_hbm.at[src_v.at[b]], kb.at[pb], semk)

        for pre in range(2):
            issue(pre, pre)

        def block_body(b, carry):
            pb = lax.rem(b, 3)
            pltpu.make_async_copy(q_hbm.at[dst_v.at[b]], qb.at[pb], semq).wait()
            pltpu.make_async_copy(k_hbm.at[src_v.at[b]], kb.at[pb], semk).wait()

            @pl.when(b + 2 < NB)
            def _():
                issue(b + 2, lax.rem(b + 2, 3))

            for g in range(G // 16):
                e16 = jnp.zeros((16,), jnp.float32)
                for j16 in range(16):
                    j = g * 16 + j16
                    acc = qb[pb, j, pl.ds(0, 16)] * kb[pb, j, pl.ds(0, 16)]
                    for c in range(1, D // 16):
                        acc += (qb[pb, j, pl.ds(c * 16, 16)]
                                * kb[pb, j, pl.ds(c * 16, 16)])
                    ej = jnp.sum(acc) * INV_SQRT_D
                    e16 = jnp.where(iota16 == j16, ej, e16)
                e_v[b, pl.ds(g * 16, 16)] = e16
                dv = dst_v[b, pl.ds(g * 16, 16)]
                cnt, _ = plsc.scan_count(dv)
                maxc = jnp.max(cnt)

                def m_pass(pp, inner):
                    mg = plsc.load_gather(m_v, [dv])
                    nm = jnp.maximum(mg, e16)
                    plsc.store_scatter(m_v, [dv], nm, mask=cnt == pp)
                    return inner

                lax.fori_loop(0, maxc + 1, m_pass, 0)
            return carry

        lax.fori_loop(0, NB, block_body, 0)

        pltpu.sync_copy(e_v, e_hbm.at[wid])
        pltpu.sync_copy(m_v, mpart_hbm.at[wid])

    return kern(q, k, src, dst)


def _merge_max_tc(mpart):
    def body(mp_ref, m_ref):
        m_ref[...] = jnp.max(mp_ref[...], axis=0, keepdims=True)

    return pl.pallas_call(
        body,
        out_shape=jax.ShapeDtypeStruct((1, N), jnp.float32),
    )(mpart)


DH = 64


def _scatter_sc(ew, m, vh, src, dst, first):
    mesh = plsc.VectorSubcoreMesh(
        core_axis_name="c", subcore_axis_name="s", num_cores=NC, num_subcores=NS
    )

    out_type = [jax.ShapeDtypeStruct((NC, N, DH), jnp.float32)]
    if first:
        out_type += [
            jax.ShapeDtypeStruct((NW, NB, G), jnp.float32),
            jax.ShapeDtypeStruct((NW, N), jnp.float32),
        ]

    @functools.partial(
        pl.kernel,
        out_type=out_type,
        mesh=mesh,
        compiler_params=pltpu.CompilerParams(
            needs_layout_passes=False, use_tc_tiling_on_sc=False),
        scratch_types=[
            pltpu.VMEM((NB, G), jnp.int32),
            pltpu.VMEM((NB, G), jnp.int32),
            pltpu.VMEM((NB, G), jnp.float32),
            pltpu.VMEM((N,), jnp.float32),
            pltpu.VMEM((N,), jnp.float32),
            pltpu.VMEM((3, G, DH), jnp.float32),
            pltpu.VMEM((G, DH), jnp.float32),
            pltpu.VMEM((208, DH), jnp.float32),
            pltpu.VMEM_SHARED((N, DH), jnp.float32),
            pltpu.SemaphoreType.DMA,
        ],
    )
    def kern(*args):
        if first:
            (ew_hbm, m_hbm, v_hbm, src_hbm, dst_hbm, s_hbm, w_hbm, dpart_hbm,
             dst_v, src_v, w_v, m_v, d_v, vb, vx, zb, s_sh, sem) = args
        else:
            (ew_hbm, v_hbm, src_hbm, dst_hbm, s_hbm,
             dst_v, src_v, w_v, m_v, d_v, vb, vx, zb, s_sh, sem) = args
        cid = lax.axis_index("c")
        sid = lax.axis_index("s")
        wid = cid * NS + sid

        pltpu.sync_copy(dst_hbm.at[wid], dst_v)
        pltpu.sync_copy(src_hbm.at[wid], src_v)
        pltpu.sync_copy(ew_hbm.at[wid], w_v)

        iota16 = lax.iota(jnp.int32, 16)
        zeros16 = jnp.zeros((16,), jnp.float32)

        if first:
            pltpu.sync_copy(m_hbm.at[0], m_v)

            def init_d(g, carry):
                d_v[pl.ds(g * 16, 16)] = zeros16
                return carry

            lax.fori_loop(0, N // 16, init_d, 0)

            def w_body(t, carry):
                b = t // (G // 16)
                g = t % (G // 16)
                dv = dst_v[b, pl.ds(g * 16, 16)]
                mg = plsc.load_gather(m_v, [dv])
                w16 = jnp.exp(w_v[b, pl.ds(g * 16, 16)] - mg)
                w_v[b, pl.ds(g * 16, 16)] = w16
                cnt, _ = plsc.scan_count(dv)
                maxc = jnp.max(cnt)

                def d_pass(pp, inner):
                    dg = plsc.load_gather(d_v, [dv])
                    plsc.store_scatter(d_v, [dv], dg + w16, mask=cnt == pp)
                    return inner

                lax.fori_loop(0, maxc + 1, d_pass, 0)
                return carry

            lax.fori_loop(0, NB * (G // 16), w_body, 0)

        def z_body(t, carry):
            zb[t // (DH // 16), pl.ds((t % (DH // 16)) * 16, 16)] = zeros16
            return carry

        lax.fori_loop(0, 208 * (DH // 16), z_body, 0)
        for r in range(RPT // 208):
            pltpu.sync_copy(zb, s_sh.at[pl.ds(sid * RPT + r * 208, 208)])

        @pl.when(sid == 0)
        def _():
            pltpu.sync_copy(zb.at[pl.ds(0, TAIL)],
                            s_sh.at[pl.ds(NS * RPT, TAIL)])

        plsc.subcore_barrier()

        pltpu.async_copy(v_hbm.at[src_v.at[0]], vb.at[0], sem)
        pltpu.async_copy(v_hbm.at[src_v.at[1]], vb.at[1], sem)

        def do_block(b, pb):
            pltpu.make_async_copy(v_hbm.at[src_v.at[b]], vb.at[pb], sem).wait()

            @pl.when(b + 2 < NB)
            def _():
                pltpu.async_copy(
                    v_hbm.at[src_v.at[b + 2]], vb.at[lax.rem(b + 2, 3)], sem)

            def group_body(g, inner):
                wg = w_v[b, pl.ds(g * 16, 16)]
                for j16 in range(16):
                    j = g * 16 + j16
                    wj = wg[j16]
                    for c in range(DH // 16):
                        vx[j, pl.ds(c * 16, 16)] = (
                            vb[pb, j, pl.ds(c * 16, 16)] * wj)
                return inner

            lax.fori_loop(0, G // 16, group_body, 0)
            pltpu.sync_copy(vx, s_sh.at[dst_v.at[b]], add=True)

        def tri_body(t, carry):
            do_block(3 * t, 0)
            do_block(3 * t + 1, 1)
            do_block(3 * t + 2, 2)
            return carry

        lax.fori_loop(0, NB // 3, tri_body, 0)
        do_block(NB - 2, 0)
        do_block(NB - 1, 1)

        plsc.subcore_barrier()

        pltpu.sync_copy(
            s_sh.at[pl.ds(sid * RPT, RPT)],
            s_hbm.at[cid].at[pl.ds(sid * RPT, RPT)],
        )

        @pl.when(sid == 0)
        def _():
            pltpu.sync_copy(s_sh.at[pl.ds(NS * RPT, TAIL)],
                            s_hbm.at[cid].at[pl.ds(NS * RPT, TAIL)])

        if first:
            pltpu.sync_copy(w_v, w_hbm.at[wid])
            pltpu.sync_copy(d_v, dpart_hbm.at[wid])

    if first:
        return kern(ew, m, vh, src, dst)
    return kern(ew, vh, src, dst)


def _merge_den_tc(dpart):
    def body(dp_ref, d_ref):
        d_ref[...] = jnp.sum(dp_ref[...], axis=0, keepdims=True)

    return pl.pallas_call(
        body,
        out_shape=jax.ShapeDtypeStruct((1, N), jnp.float32),
    )(dpart)


def _finish_tc(x, S0, S1, den, Wo, g2, b2):
    R = 1000

    def body(x_ref, s0_ref, s1_ref, den_ref, wo_ref, g_ref, b_ref, y_ref):
        rden = 1.0 / (den_ref[...] + 1e-16)
        aggl = (s0_ref[0] + s0_ref[1]) * rden
        aggr = (s1_ref[0] + s1_ref[1]) * rden
        out = (
            jnp.dot(aggl, wo_ref[pl.ds(0, DH), :],
                    preferred_element_type=jnp.float32)
            + jnp.dot(aggr, wo_ref[pl.ds(DH, DH), :],
                      preferred_element_type=jnp.float32)
        )
        mu = jnp.mean(out, axis=-1, keepdims=True)
        var = jnp.mean((out - mu) ** 2, axis=-1, keepdims=True)
        ln = (out - mu) / jnp.sqrt(var + 1e-5) * g_ref[...] + b_ref[...]
        y_ref[...] = x_ref[...] + jnp.maximum(ln, 0.0)

    return pl.pallas_call(
        body,
        grid=(N // R,),
        in_specs=[
            pl.BlockSpec((R, D), lambda i: (i, 0)),
            pl.BlockSpec((NC, R, DH), lambda i: (0, i, 0)),
            pl.BlockSpec((NC, R, DH), lambda i: (0, i, 0)),
            pl.BlockSpec((R, 1), lambda i: (i, 0)),
            pl.BlockSpec((D, D), lambda i: (0, 0)),
            pl.BlockSpec((1, D), lambda i: (0, 0)),
            pl.BlockSpec((1, D), lambda i: (0, 0)),
        ],
        out_specs=pl.BlockSpec((R, D), lambda i: (i, 0)),
        out_shape=jax.ShapeDtypeStruct((N, D), jnp.float32),
    )(x, S0, S1, den, Wo, g2.reshape(1, D), b2.reshape(1, D))


def kernel(x, edge_index, Wq, Wk, Wv, Wo, g1, b1, g2, b2):
    src = edge_index[0].reshape(NW, NB, G)
    dst = edge_index[1].reshape(NW, NB, G)
    q, k, vl, vr = _qkv_tc(x, Wq, Wk, Wv, g1, b1)
    e, mpart = _edge_logits_sc(q, k, src, dst)
    m = _merge_max_tc(mpart)
    S0, w, dpart = _scatter_sc(e, m, vl, src, dst, first=True)
    (S1,) = _scatter_sc(w, None, vr, src, dst, first=False)
    den = _merge_den_tc(dpart).reshape(N, 1)
    return _finish_tc(x, S0, S1, den, Wo, g2, b2)


# speedup vs baseline: 2.5062x; 1.0094x over previous
import functools

import jax
import jax.numpy as jnp
from jax import lax
from jax.experimental import pallas as pl
from jax.experimental.pallas import tpu as pltpu
from jax.experimental.pallas import tpu_sc as plsc

N = 10000
E = 320000
D = 128
NC = 2
NS = 16
NW = NC * NS
EC = E // NW
G = 80
NB = EC // G
RPT = 624
TAIL = N - NS * RPT
INV_SQRT_D = 1.0 / float(D) ** 0.5


def _qkv_tc(x, Wq, Wk, Wv, g1, b1):
    R = 2000

    def body(x_ref, wq_ref, wk_ref, wv_ref, g_ref, b_ref, q_ref, k_ref, vl_ref, vr_ref):
        xb = x_ref[...]
        mu = jnp.mean(xb, axis=-1, keepdims=True)
        var = jnp.mean((xb - mu) ** 2, axis=-1, keepdims=True)
        h = (xb - mu) / jnp.sqrt(var + 1e-5) * g_ref[...] + b_ref[...]
        q_ref[...] = jnp.dot(h, wq_ref[...], preferred_element_type=jnp.float32)
        k_ref[...] = jnp.dot(h, wk_ref[...], preferred_element_type=jnp.float32)
        vfull = jnp.dot(h, wv_ref[...], preferred_element_type=jnp.float32)
        vl_ref[...] = vfull[:, :DH]
        vr_ref[...] = vfull[:, DH:]

    out = pl.pallas_call(
        body,
        grid=(N // R,),
        in_specs=[
            pl.BlockSpec((R, D), lambda i: (i, 0)),
            pl.BlockSpec((D, D), lambda i: (0, 0)),
            pl.BlockSpec((D, D), lambda i: (0, 0)),
            pl.BlockSpec((D, D), lambda i: (0, 0)),
            pl.BlockSpec((1, D), lambda i: (0, 0)),
            pl.BlockSpec((1, D), lambda i: (0, 0)),
        ],
        out_specs=[
            pl.BlockSpec((R, D), lambda i: (i, 0)),
            pl.BlockSpec((R, D), lambda i: (i, 0)),
            pl.BlockSpec((R, DH), lambda i: (i, 0)),
            pl.BlockSpec((R, DH), lambda i: (i, 0)),
        ],
        out_shape=[jax.ShapeDtypeStruct((N, D), jnp.float32)] * 2
        + [jax.ShapeDtypeStruct((N, DH), jnp.float32)] * 2,
    )(x, Wq, Wk, Wv, g1.reshape(1, D), b1.reshape(1, D))
    return out


def _edge_logits_sc(q, k, src, dst):
    mesh = plsc.VectorSubcoreMesh(
        core_axis_name="c", subcore_axis_name="s", num_cores=NC, num_subcores=NS
    )

    @functools.partial(
        pl.kernel,
        out_type=[
            jax.ShapeDtypeStruct((NW, NB, G), jnp.float32),
            jax.ShapeDtypeStruct((NW, N), jnp.float32),
        ],
        mesh=mesh,
        compiler_params=pltpu.CompilerParams(needs_layout_passes=False),
        scratch_types=[
            pltpu.VMEM((NB, G), jnp.int32),
            pltpu.VMEM((NB, G), jnp.int32),
            pltpu.VMEM((NB, G), jnp.float32),
            pltpu.VMEM((N,), jnp.float32),
            pltpu.VMEM((3, G, D), jnp.float32),
            pltpu.VMEM((3, G, D), jnp.float32),
            pltpu.SemaphoreType.DMA,
            pltpu.SemaphoreType.DMA,
        ],
    )
    def kern(q_hbm, k_hbm, src_hbm, dst_hbm, e_hbm, mpart_hbm,
             dst_v, src_v, e_v, m_v, qb, kb, semq, semk):
        cid = lax.axis_index("c")
        sid = lax.axis_index("s")
        wid = cid * NS + sid

        pltpu.sync_copy(dst_hbm.at[wid], dst_v)
        pltpu.sync_copy(src_hbm.at[wid], src_v)

        neg = jnp.full((16,), -1e30, dtype=jnp.float32)

        def init_m(g, carry):
            m_v[pl.ds(g * 16, 16)] = neg
            return carry

        lax.fori_loop(0, N // 16, init_m, 0)

        iota16 = lax.iota(jnp.int32, 16)

        def issue(b, pb):
            pltpu.async_copy(q_hbm.at[dst_v.at[b]], qb.at[pb], semq)
            pltpu.async_copy(k_hbm.at[src_v.at[b]], kb.at[pb], semk)

        for pre in range(2):
            issue(pre, pre)

        def block_body(b, carry):
            pb = lax.rem(b, 3)
            pltpu.make_async_copy(q_hbm.at[dst_v.at[b]], qb.at[pb], semq).wait()
            pltpu.make_async_copy(k_hbm.at[src_v.at[b]], kb.at[pb], semk).wait()

            @pl.when(b + 2 < NB)
            def _():
                issue(b + 2, lax.rem(b + 2, 3))

            for g in range(G // 16):
                e16 = jnp.zeros((16,), jnp.float32)
                for j16 in range(16):
                    j = g * 16 + j16
                    acc = qb[pb, j, pl.ds(0, 16)] * kb[pb, j, pl.ds(0, 16)]
                    for c in range(1, D // 16):
                        acc += (qb[pb, j, pl.ds(c * 16, 16)]
                                * kb[pb, j, pl.ds(c * 16, 16)])
                    ej = jnp.sum(acc) * INV_SQRT_D
                    e16 = jnp.where(iota16 == j16, ej, e16)
                e_v[b, pl.ds(g * 16, 16)] = e16
                dv = dst_v[b, pl.ds(g * 16, 16)]
                cnt, _ = plsc.scan_count(dv)
                maxc = jnp.max(cnt)

                def m_pass(pp, inner):
                    mg = plsc.load_gather(m_v, [dv])
                    nm = jnp.maximum(mg, e16)
                    plsc.store_scatter(m_v, [dv], nm, mask=cnt == pp)
                    return inner

                lax.fori_loop(0, maxc + 1, m_pass, 0)
            return carry

        lax.fori_loop(0, NB, block_body, 0)

        pltpu.sync_copy(e_v, e_hbm.at[wid])
        pltpu.sync_copy(m_v, mpart_hbm.at[wid])

    return kern(q, k, src, dst)


def _merge_max_tc(mpart):
    def body(mp_ref, m_ref):
        m_ref[...] = jnp.max(mp_ref[...], axis=0, keepdims=True)

    return pl.pallas_call(
        body,
        out_shape=jax.ShapeDtypeStruct((1, N), jnp.float32),
    )(mpart)


DH = 64


def _scatter_sc(ew, m, vh, src, dst, first):
    mesh = plsc.VectorSubcoreMesh(
        core_axis_name="c", subcore_axis_name="s", num_cores=NC, num_subcores=NS
    )

    out_type = [jax.ShapeDtypeStruct((NC, N, DH), jnp.float32)]
    if first:
        out_type += [
            jax.ShapeDtypeStruct((NW, NB, G), jnp.float32),
            jax.ShapeDtypeStruct((NW, N), jnp.float32),
        ]

    @functools.partial(
        pl.kernel,
        out_type=out_type,
        mesh=mesh,
        compiler_params=pltpu.CompilerParams(
            needs_layout_passes=False, use_tc_tiling_on_sc=False),
        scratch_types=[
            pltpu.VMEM((NB, G), jnp.int32),
            pltpu.VMEM((NB, G), jnp.int32),
            pltpu.VMEM((NB, G), jnp.float32),
            pltpu.VMEM((N,), jnp.float32),
            pltpu.VMEM((N,), jnp.float32),
            pltpu.VMEM((3, G, DH), jnp.float32),
            pltpu.VMEM((G, DH), jnp.float32),
            pltpu.VMEM((208, DH), jnp.float32),
            pltpu.VMEM_SHARED((N, DH), jnp.float32),
            pltpu.SemaphoreType.DMA,
        ],
    )
    def kern(*args):
        if first:
            (ew_hbm, m_hbm, v_hbm, src_hbm, dst_hbm, s_hbm, w_hbm, dpart_hbm,
             dst_v, src_v, w_v, m_v, d_v, vb, vx, zb, s_sh, sem) = args
        else:
            (ew_hbm, v_hbm, src_hbm, dst_hbm, s_hbm,
             dst_v, src_v, w_v, m_v, d_v, vb, vx, zb, s_sh, sem) = args
        cid = lax.axis_index("c")
        sid = lax.axis_index("s")
        wid = cid * NS + sid

        pltpu.sync_copy(dst_hbm.at[wid], dst_v)
        pltpu.sync_copy(src_hbm.at[wid], src_v)
        pltpu.sync_copy(ew_hbm.at[wid], w_v)

        iota16 = lax.iota(jnp.int32, 16)
        zeros16 = jnp.zeros((16,), jnp.float32)

        if first:
            pltpu.sync_copy(m_hbm.at[0], m_v)

            def init_d(g, carry):
                d_v[pl.ds(g * 16, 16)] = zeros16
                return carry

            lax.fori_loop(0, N // 16, init_d, 0)

            def w_body(t, carry):
                b = t // (G // 16)
                g = t % (G // 16)
                dv = dst_v[b, pl.ds(g * 16, 16)]
                mg = plsc.load_gather(m_v, [dv])
                w16 = jnp.exp(w_v[b, pl.ds(g * 16, 16)] - mg)
                w_v[b, pl.ds(g * 16, 16)] = w16
                cnt, _ = plsc.scan_count(dv)
                maxc = jnp.max(cnt)

                def d_pass(pp, inner):
                    dg = plsc.load_gather(d_v, [dv])
                    plsc.store_scatter(d_v, [dv], dg + w16, mask=cnt == pp)
                    return inner

                lax.fori_loop(0, maxc + 1, d_pass, 0)
                return carry

            lax.fori_loop(0, NB * (G // 16), w_body, 0)

        def z_body(t, carry):
            zb[t // (DH // 16), pl.ds((t % (DH // 16)) * 16, 16)] = zeros16
            return carry

        lax.fori_loop(0, 208 * (DH // 16), z_body, 0)
        for r in range(RPT // 208):
            pltpu.sync_copy(zb, s_sh.at[pl.ds(sid * RPT + r * 208, 208)])

        @pl.when(sid == 0)
        def _():
            pltpu.sync_copy(zb.at[pl.ds(0, TAIL)],
                            s_sh.at[pl.ds(NS * RPT, TAIL)])

        plsc.subcore_barrier()

        pltpu.async_copy(v_hbm.at[src_v.at[0]], vb.at[0], sem)
        pltpu.async_copy(v_hbm.at[src_v.at[1]], vb.at[1], sem)

        def do_block(b, pb):
            pltpu.make_async_copy(v_hbm.at[src_v.at[b]], vb.at[pb], sem).wait()

            @pl.when(b + 2 < NB)
            def _():
                pltpu.async_copy(
                    v_hbm.at[src_v.at[b + 2]], vb.at[lax.rem(b + 2, 3)], sem)

            def group_body(g, inner):
                wg = w_v[b, pl.ds(g * 16, 16)]
                for j16 in range(16):
                    j = g * 16 + j16
                    wj = wg[j16]
                    for c in range(DH // 16):
                        vx[j, pl.ds(c * 16, 16)] = (
                            vb[pb, j, pl.ds(c * 16, 16)] * wj)
                return inner

            lax.fori_loop(0, G // 16, group_body, 0)
            pltpu.sync_copy(vx, s_sh.at[dst_v.at[b]], add=True)

        def tri_body(t, carry):
            do_block(3 * t, 0)
            do_block(3 * t + 1, 1)
            do_block(3 * t + 2, 2)
            return carry

        lax.fori_loop(0, NB // 3, tri_body, 0)
        do_block(NB - 2, 0)
        do_block(NB - 1, 1)

        plsc.subcore_barrier()

        pltpu.sync_copy(
            s_sh.at[pl.ds(sid * RPT, RPT)],
            s_hbm.at[cid].at[pl.ds(sid * RPT, RPT)],
        )

        @pl.when(sid == 0)
        def _():
            pltpu.sync_copy(s_sh.at[pl.ds(NS * RPT, TAIL)],
                            s_hbm.at[cid].at[pl.ds(NS * RPT, TAIL)])

        if first:
            pltpu.sync_copy(w_v, w_hbm.at[wid])
            pltpu.sync_copy(d_v, dpart_hbm.at[wid])

    if first:
        return kern(ew, m, vh, src, dst)
    return kern(ew, vh, src, dst)


def _merge_den_tc(dpart):
    def body(dp_ref, d_ref):
        d_ref[...] = jnp.sum(dp_ref[...], axis=0, keepdims=True)

    return pl.pallas_call(
        body,
        out_shape=jax.ShapeDtypeStruct((1, N), jnp.float32),
    )(dpart)


def _finish_tc(x, S0, S1, den, Wo, g2, b2):
    R = 2000

    def body(x_ref, s0_ref, s1_ref, den_ref, wo_ref, g_ref, b_ref, y_ref):
        rden = 1.0 / (den_ref[...] + 1e-16)
        aggl = (s0_ref[0] + s0_ref[1]) * rden
        aggr = (s1_ref[0] + s1_ref[1]) * rden
        out = (
            jnp.dot(aggl, wo_ref[pl.ds(0, DH), :],
                    preferred_element_type=jnp.float32)
            + jnp.dot(aggr, wo_ref[pl.ds(DH, DH), :],
                      preferred_element_type=jnp.float32)
        )
        mu = jnp.mean(out, axis=-1, keepdims=True)
        var = jnp.mean((out - mu) ** 2, axis=-1, keepdims=True)
        ln = (out - mu) / jnp.sqrt(var + 1e-5) * g_ref[...] + b_ref[...]
        y_ref[...] = x_ref[...] + jnp.maximum(ln, 0.0)

    return pl.pallas_call(
        body,
        grid=(N // R,),
        in_specs=[
            pl.BlockSpec((R, D), lambda i: (i, 0)),
            pl.BlockSpec((NC, R, DH), lambda i: (0, i, 0)),
            pl.BlockSpec((NC, R, DH), lambda i: (0, i, 0)),
            pl.BlockSpec((R, 1), lambda i: (i, 0)),
            pl.BlockSpec((D, D), lambda i: (0, 0)),
            pl.BlockSpec((1, D), lambda i: (0, 0)),
            pl.BlockSpec((1, D), lambda i: (0, 0)),
        ],
        out_specs=pl.BlockSpec((R, D), lambda i: (i, 0)),
        out_shape=jax.ShapeDtypeStruct((N, D), jnp.float32),
    )(x, S0, S1, den, Wo, g2.reshape(1, D), b2.reshape(1, D))


def kernel(x, edge_index, Wq, Wk, Wv, Wo, g1, b1, g2, b2):
    src = edge_index[0].reshape(NW, NB, G)
    dst = edge_index[1].reshape(NW, NB, G)
    q, k, vl, vr = _qkv_tc(x, Wq, Wk, Wv, g1, b1)
    e, mpart = _edge_logits_sc(q, k, src, dst)
    m = _merge_max_tc(mpart)
    S0, w, dpart = _scatter_sc(e, m, vl, src, dst, first=True)
    (S1,) = _scatter_sc(w, None, vr, src, dst, first=False)
    den = _merge_den_tc(dpart).reshape(N, 1)
    return _finish_tc(x, S0, S1, den, Wo, g2, b2)
